# baseline (speedup 1.0000x reference)
import functools

import jax
import jax.numpy as jnp
from jax import lax
from jax.experimental import pallas as pl
from jax.experimental.pallas import tpu as pltpu
from jax.experimental.pallas import tpu_sc as plsc

NC = 2
NS = 16
L = 16
CHUNK = 128
GC = 80
DH = 128
PW = DH // 2

_PERM = [0] * DH
for _g in range(DH // 32):
    for _k in range(L):
        _PERM[32 * _g + _k] = 32 * _g + 2 * _k
        _PERM[32 * _g + L + _k] = 32 * _g + 2 * _k + 1


def _round_up(a, b):
    return (a + b - 1) // b * b


def _sc_segment_sum(xpk, srcoff, dstr, npad, nchunk):
    mesh = plsc.VectorSubcoreMesh(core_axis_name="c", subcore_axis_name="s")
    rows_per_tile = npad // NS
    nzero = rows_per_tile // GC

    @functools.partial(
        pl.kernel,
        out_type=jax.ShapeDtypeStruct((NC, npad, DH), jnp.float32),
        mesh=mesh,
        scratch_types=[
            pltpu.VMEM((nchunk, GC), jnp.int32),
            pltpu.VMEM((1, GC), jnp.int32),
            pltpu.VMEM((1, GC), jnp.int32),
            pltpu.VMEM((GC, PW), jnp.int32),
            pltpu.VMEM((GC, PW), jnp.int32),
            pltpu.VMEM((GC, DH), jnp.float32),
            pltpu.VMEM((GC, DH), jnp.float32),
            pltpu.VMEM_SHARED((npad, DH), jnp.float32),
            pltpu.SemaphoreType.DMA,
            pltpu.SemaphoreType.DMA,
            pltpu.SemaphoreType.DMA,
        ],
        compiler_params=pltpu.CompilerParams(use_tc_tiling_on_sc=False),
    )
    def k(xpk_h, srcoff_h, dstr_h, out_h, srcb, dstb0, dstb1, raw0, raw1,
          fbuf0, fbuf1, acc, semg, semd, semsc):
        c = lax.axis_index("c")
        s = lax.axis_index("s")

        def fill_g(i, _):
            for l in range(DH // L):
                fbuf0[i, pl.ds(l * L, L)] = jnp.zeros((L,), jnp.float32)
            return 0
        lax.fori_loop(0, GC, fill_g, 0)

        def zero_acc(j, _):
            base = s * rows_per_tile + j * GC
            pltpu.sync_copy(fbuf0, acc.at[pl.ds(base, GC)])
            return 0
        lax.fori_loop(0, nzero, zero_acc, 0)

        plsc.subcore_barrier()

        pltpu.sync_copy(srcoff_h.at[c, s], srcb)

        def wait_g(buf):
            pltpu.make_async_copy(xpk_h.at[pl.ds(0, GC)], buf, semg).wait()

        def wait_d(buf):
            pltpu.make_async_copy(
                dstr_h.at[s, pl.ds(0, 1)], buf, semd).wait()

        def wait_sc(buf):
            pltpu.make_async_copy(
                buf, acc.at[pl.ds(0, GC)], semsc).wait()

        def convert(raw, fbuf):
            def cv(i2, _):
                for r in range(8):
                    i = 8 * i2 + r
                    for g in range(PW // L):
                        w = raw[i, pl.ds(g * L, L)]
                        a0 = lax.bitcast_convert_type(
                            lax.shift_left(w, 16), jnp.float32)
                        a1 = lax.bitcast_convert_type(
                            w & jnp.int32(-65536), jnp.float32)
                        fbuf[i, pl.ds(32 * g, L)] = a0
                        fbuf[i, pl.ds(32 * g + L, L)] = a1
                return 0
            lax.fori_loop(0, GC // 8, cv, 0)

        def stage(jq, raw, fbuf, dstb, first):
            wait_g(raw)
            if not first:
                wait_sc(fbuf)
            pltpu.async_copy(dstr_h.at[s, pl.ds(jq, 1)], dstb, semd)
            convert(raw, fbuf)

            @pl.when(jq + 2 < nchunk)
            def _():
                pltpu.async_copy(xpk_h.at[srcb.at[jq + 2]], raw, semg)
            wait_d(dstb)
            pltpu.async_copy(fbuf, acc.at[dstb.at[0]], semsc, add=True)

        pltpu.async_copy(xpk_h.at[srcb.at[0]], raw0, semg)
        pltpu.async_copy(xpk_h.at[srcb.at[1]], raw1, semg)

        stage(0, raw0, fbuf0, dstb0, True)
        stage(1, raw1, fbuf1, dstb1, True)

        def body(jp, _):
            j0 = 2 * jp + 2
            stage(j0, raw0, fbuf0, dstb0, False)
            stage(j0 + 1, raw1, fbuf1, dstb1, False)
            return 0
        lax.fori_loop(0, nchunk // 2 - 1, body, 0)

        wait_sc(fbuf0)
        wait_sc(fbuf1)

        plsc.subcore_barrier()

        base = s * rows_per_tile
        pltpu.sync_copy(acc.at[pl.ds(base, rows_per_tile)],
                        out_h.at[c, pl.ds(base, rows_per_tile)])

    return k(xpk, srcoff, dstr)


def _sc_counts(dstr2, npad, nchunk2):
    mesh = plsc.VectorSubcoreMesh(core_axis_name="c", subcore_axis_name="s")
    rows_per_tile = npad // NS
    nzero = rows_per_tile // CHUNK

    @functools.partial(
        pl.kernel,
        out_type=jax.ShapeDtypeStruct((NC, npad, L), jnp.float32),
        mesh=mesh,
        scratch_types=[
            pltpu.VMEM((nchunk2, CHUNK), jnp.int32),
            pltpu.VMEM((CHUNK, L), jnp.float32),
            pltpu.VMEM_SHARED((npad, L), jnp.float32),
            pltpu.SemaphoreType.DMA,
        ],
        compiler_params=pltpu.CompilerParams(use_tc_tiling_on_sc=False),
    )
    def k(dstr_h, cnt_h, dstb, onesb, cacc, sem):
        c = lax.axis_index("c")
        s = lax.axis_index("s")

        def fill(i, _):
            onesb[i, :] = jnp.zeros((L,), jnp.float32)
            return 0
        lax.fori_loop(0, CHUNK, fill, 0)

        def zero_acc(j, _):
            base = s * rows_per_tile + j * CHUNK
            pltpu.sync_copy(onesb, cacc.at[pl.ds(base, CHUNK)])
            return 0
        lax.fori_loop(0, nzero, zero_acc, 0)

        def fill1(i, _):
            onesb[i, :] = jnp.ones((L,), jnp.float32)
            return 0
        lax.fori_loop(0, CHUNK, fill1, 0)

        plsc.subcore_barrier()

        pltpu.sync_copy(dstr_h.at[c, s], dstb)

        def body(j, _):
            pltpu.sync_copy(onesb, cacc.at[dstb.at[j]], add=True)
            return 0
        lax.fori_loop(0, nchunk2, body, 0)

        plsc.subcore_barrier()

        base = s * rows_per_tile
        pltpu.sync_copy(cacc.at[pl.ds(base, rows_per_tile)],
                        cnt_h.at[c, pl.ds(base, rows_per_tile)])

    return k(dstr2)


def _tc_head(summed, counts, x, wlt0, wlt1, bl, wrt, w1t, b1, w2t, b2,
             w3t, b3):
    n, d = x.shape
    blk = 2000

    def body(s0_r, s1_r, c0_r, c1_r, x_r, wlt0_r, wlt1_r, bl_r, wrt_r,
             w1t_r, b1_r, w2t_r, b2_r, w3t_r, b3_r, out_r):
        cnt = c0_r[0][:, 0:1] + c1_r[0][:, 0:1]
        rcp = 1.0 / jnp.maximum(cnt, 1.0)
        m0 = s0_r[0] * rcp
        m1 = s1_r[0] * rcp
        xv = x_r[...]
        conv = (jnp.dot(m0, wlt0_r[...], preferred_element_type=jnp.float32)
                + jnp.dot(m1, wlt1_r[...], preferred_element_type=jnp.float32)
                + bl_r[...]
                + jnp.dot(xv, wrt_r[...], preferred_element_type=jnp.float32))
        h = jnp.maximum(conv, 0.0) + xv
        h = jnp.maximum(jnp.dot(h, w1t_r[...], preferred_element_type=jnp.float32)
                        + b1_r[...], 0.0)
        h = jnp.maximum(jnp.dot(h, w2t_r[...], preferred_element_type=jnp.float32)
                        + b2_r[...], 0.0)
        out_r[...] = jnp.dot(h, w3t_r[...],
                             preferred_element_type=jnp.float32) + b3_r[...]

    full = lambda shape: pl.BlockSpec(shape, lambda i: (0, 0))
    return pl.pallas_call(
        body,
        grid=(n // blk,),
        in_specs=[
            pl.BlockSpec((1, blk, DH), lambda i: (0, i, 0)),
            pl.BlockSpec((1, blk, DH), lambda i: (1, i, 0)),
            pl.BlockSpec((1, blk, L), lambda i: (0, i, 0)),
            pl.BlockSpec((1, blk, L), lambda i: (1, i, 0)),
            pl.BlockSpec((blk, d), lambda i: (i, 0)),
            full(wlt0.shape), full(wlt1.shape), full(bl.shape), full(wrt.shape),
            full(w1t.shape), full(b1.shape), full(w2t.shape), full(b2.shape),
            full(w3t.shape), full(b3.shape),
        ],
        out_specs=pl.BlockSpec((blk, 1), lambda i: (i, 0)),
        out_shape=jax.ShapeDtypeStruct((n, 1), jnp.float32),
    )(summed, summed, counts, counts, x, wlt0, wlt1, bl, wrt,
      w1t, b1, w2t, b2, w3t, b3)


def kernel(x, edge_index, Wl, bl, Wr, W1, b1, W2, b2, W3, b3):
    n, d = x.shape
    e = edge_index.shape[1]
    npad = _round_up(n + 1, NS * CHUNK)
    epad = _round_up(e, 2 * NS * GC)
    nchunk = epad // (NS * GC)
    epad2 = _round_up(e, NC * NS * CHUNK)
    nchunk2 = epad2 // (NC * NS * CHUNK)

    src = edge_index[0]
    dst = edge_index[1]

    xh0 = jnp.pad(x[:, :DH], ((0, npad - n), (0, 0)))
    xh1 = jnp.pad(x[:, DH:], ((0, npad - n), (0, 0)))
    xcomb = jnp.concatenate([xh0, xh1], axis=0)
    u = lax.bitcast_convert_type(xcomb, jnp.uint32)
    r16 = (u + 0x7FFF + ((u >> 16) & 1)) >> 16
    lo = r16[:, 0::2]
    hi = r16[:, 1::2]
    xpk = lax.bitcast_convert_type(lo | (hi << 16), jnp.int32)

    srcp = jnp.pad(src, (0, epad - e))
    dstp = jnp.pad(dst, (0, epad - e), constant_values=n)
    srcoff = jnp.stack([srcp, srcp + npad]).reshape(NC, NS, nchunk, GC)
    dstr = dstp.reshape(NS, nchunk, GC)

    dstp2 = jnp.pad(dst, (0, epad2 - e), constant_values=n)
    dstr2 = dstp2.reshape(NC, NS, nchunk2, CHUNK)

    summed = _sc_segment_sum(xpk, srcoff, dstr, npad, nchunk)
    counts = _sc_counts(dstr2, npad, nchunk2)

    perm = jnp.array(_PERM, dtype=jnp.int32)
    wlt = Wl.T
    out = _tc_head(
        summed, counts, x,
        wlt[perm], wlt[DH + perm], bl.reshape(1, -1), Wr.T,
        W1.T, b1.reshape(1, -1), W2.T, b2.reshape(1, -1),
        W3.T, b3.reshape(1, -1),
    )
    return out


# speedup vs baseline: 2.3038x; 2.3038x over previous
import functools

import jax
import jax.numpy as jnp
from jax import lax
from jax.experimental import pallas as pl
from jax.experimental.pallas import tpu as pltpu
from jax.experimental.pallas import tpu_sc as plsc

NC = 2
NS = 16
L = 16
CHUNK = 128
GC = 120
DH = 128
PW = DH // 2


def _round_up(a, b):
    return (a + b - 1) // b * b


def _sc_segment_sum(xpk, srcoff, dstr, npad, nchunk):
    mesh = plsc.VectorSubcoreMesh(core_axis_name="c", subcore_axis_name="s")
    rows_per_tile = npad // NS
    nzero = rows_per_tile // GC

    @functools.partial(
        pl.kernel,
        out_type=jax.ShapeDtypeStruct((NC, npad, DH), jnp.float32),
        mesh=mesh,
        scratch_types=[
            pltpu.VMEM((1, GC), jnp.int32),
            pltpu.VMEM((1, GC), jnp.int32),
            pltpu.VMEM((1, GC), jnp.int32),
            pltpu.VMEM((1, GC), jnp.int32),
            pltpu.VMEM((GC, PW), jnp.int32),
            pltpu.VMEM((GC, PW), jnp.int32),
            pltpu.VMEM((GC, DH), jnp.float32),
            pltpu.VMEM((GC, DH), jnp.float32),
            pltpu.VMEM_SHARED((npad, DH), jnp.float32),
            pltpu.SemaphoreType.DMA,
            pltpu.SemaphoreType.DMA,
            pltpu.SemaphoreType.DMA,
            pltpu.SemaphoreType.DMA,
        ],
        compiler_params=pltpu.CompilerParams(use_tc_tiling_on_sc=False),
    )
    def k(xpk_h, srcoff_h, dstr_h, out_h, srcs0, srcs1, dstb0, dstb1,
          raw0, raw1, fbuf0, fbuf1, acc, semg, semd, semsc, sems):
        c = lax.axis_index("c")
        s = lax.axis_index("s")

        def fill_g(i, _):
            for l in range(DH // L):
                fbuf0[i, pl.ds(l * L, L)] = jnp.zeros((L,), jnp.float32)
            return 0
        lax.fori_loop(0, GC, fill_g, 0)

        def zero_acc(j, _):
            base = s * rows_per_tile + j * GC
            pltpu.sync_copy(fbuf0, acc.at[pl.ds(base, GC)])
            return 0
        lax.fori_loop(0, nzero, zero_acc, 0)
        rem = rows_per_tile - nzero * GC
        if rem:
            pltpu.sync_copy(
                fbuf0.at[pl.ds(0, rem)],
                acc.at[pl.ds(s * rows_per_tile + nzero * GC, rem)])

        plsc.subcore_barrier()

        def wait_g(buf):
            pltpu.make_async_copy(xpk_h.at[pl.ds(0, GC)], buf, semg).wait()

        def wait_s(buf):
            pltpu.make_async_copy(
                srcoff_h.at[0, 0, pl.ds(0, 1)], buf, sems).wait()

        def wait_d(buf):
            pltpu.make_async_copy(
                dstr_h.at[0, pl.ds(0, 1)], buf, semd).wait()

        def wait_sc(buf):
            pltpu.make_async_copy(
                buf, acc.at[pl.ds(0, GC)], semsc).wait()

        def fire_src(j, sbuf):
            pltpu.async_copy(srcoff_h.at[c, s, pl.ds(j, 1)], sbuf, sems)

        def convert(raw, fbuf):
            def cv(i2, _):
                for r in range(4):
                    i = 4 * i2 + r
                    for g in range(PW // L):
                        w = raw[i, pl.ds(g * L, L)]
                        a0 = lax.bitcast_convert_type(
                            lax.shift_left(w, 16), jnp.float32)
                        a1 = lax.bitcast_convert_type(
                            w & jnp.int32(-65536), jnp.float32)
                        fbuf[i, pl.ds(L * g, L)] = a0
                        fbuf[i, pl.ds(PW + L * g, L)] = a1
                return 0
            lax.fori_loop(0, GC // 4, cv, 0)

        def stage(jq, srcs, raw, fbuf, dstb, first):
            wait_g(raw)
            if not first:
                wait_sc(fbuf)

            @pl.when(jq + 2 < nchunk)
            def _():
                fire_src(jq + 2, srcs)
            pltpu.async_copy(dstr_h.at[s, pl.ds(jq, 1)], dstb, semd)
            convert(raw, fbuf)

            @pl.when(jq + 2 < nchunk)
            def _():
                wait_s(srcs)
                pltpu.async_copy(xpk_h.at[srcs.at[0]], raw, semg)
            wait_d(dstb)
            pltpu.async_copy(fbuf, acc.at[dstb.at[0]], semsc, add=True)

        fire_src(0, srcs0)
        fire_src(1, srcs1)
        wait_s(srcs0)
        pltpu.async_copy(xpk_h.at[srcs0.at[0]], raw0, semg)
        wait_s(srcs1)
        pltpu.async_copy(xpk_h.at[srcs1.at[0]], raw1, semg)

        stage(0, srcs0, raw0, fbuf0, dstb0, True)
        stage(1, srcs1, raw1, fbuf1, dstb1, True)

        def body(jp, _):
            j0 = 2 * jp + 2
            stage(j0, srcs0, raw0, fbuf0, dstb0, False)
            stage(j0 + 1, srcs1, raw1, fbuf1, dstb1, False)
            return 0
        lax.fori_loop(0, nchunk // 2 - 1, body, 0)

        wait_sc(fbuf0)
        wait_sc(fbuf1)

        plsc.subcore_barrier()

        base = s * rows_per_tile
        pltpu.sync_copy(acc.at[pl.ds(base, rows_per_tile)],
                        out_h.at[c, pl.ds(base, rows_per_tile)])

    return k(xpk, srcoff, dstr)


def _sc_counts(dstr2, npad, nchunk2):
    mesh = plsc.VectorSubcoreMesh(core_axis_name="c", subcore_axis_name="s")
    rows_per_tile = npad // NS
    nzero = rows_per_tile // CHUNK

    @functools.partial(
        pl.kernel,
        out_type=jax.ShapeDtypeStruct((NC, npad, L), jnp.float32),
        mesh=mesh,
        scratch_types=[
            pltpu.VMEM((nchunk2, CHUNK), jnp.int32),
            pltpu.VMEM((CHUNK, L), jnp.float32),
            pltpu.VMEM_SHARED((npad, L), jnp.float32),
            pltpu.SemaphoreType.DMA,
        ],
        compiler_params=pltpu.CompilerParams(use_tc_tiling_on_sc=False),
    )
    def k(dstr_h, cnt_h, dstb, onesb, cacc, sem):
        c = lax.axis_index("c")
        s = lax.axis_index("s")

        def fill(i, _):
            onesb[i, :] = jnp.zeros((L,), jnp.float32)
            return 0
        lax.fori_loop(0, CHUNK, fill, 0)

        def zero_acc(j, _):
            base = s * rows_per_tile + j * CHUNK
            pltpu.sync_copy(onesb, cacc.at[pl.ds(base, CHUNK)])
            return 0
        lax.fori_loop(0, nzero, zero_acc, 0)

        def fill1(i, _):
            onesb[i, :] = jnp.ones((L,), jnp.float32)
            return 0
        lax.fori_loop(0, CHUNK, fill1, 0)

        plsc.subcore_barrier()

        pltpu.sync_copy(dstr_h.at[c, s], dstb)

        def body(j, _):
            pltpu.sync_copy(onesb, cacc.at[dstb.at[j]], add=True)
            return 0
        lax.fori_loop(0, nchunk2, body, 0)

        plsc.subcore_barrier()

        base = s * rows_per_tile
        pltpu.sync_copy(cacc.at[pl.ds(base, rows_per_tile)],
                        cnt_h.at[c, pl.ds(base, rows_per_tile)])

    return k(dstr2)


def _tc_head(summed, counts, x, wlt0, wlt1, bl, wrt, w1t, b1, w2t, b2,
             w3t, b3):
    n, d = x.shape
    blk = 2000

    def body(s0_r, s1_r, c0_r, c1_r, x_r, wlt0_r, wlt1_r, bl_r, wrt_r,
             w1t_r, b1_r, w2t_r, b2_r, w3t_r, b3_r, out_r):
        cnt = c0_r[0][:, 0:1] + c1_r[0][:, 0:1]
        rcp = 1.0 / jnp.maximum(cnt, 1.0)
        m0 = s0_r[0] * rcp
        m1 = s1_r[0] * rcp
        xv = x_r[...]
        conv = (jnp.dot(m0, wlt0_r[...], preferred_element_type=jnp.float32)
                + jnp.dot(m1, wlt1_r[...], preferred_element_type=jnp.float32)
                + bl_r[...]
                + jnp.dot(xv, wrt_r[...], preferred_element_type=jnp.float32))
        h = jnp.maximum(conv, 0.0) + xv
        h = jnp.maximum(jnp.dot(h, w1t_r[...], preferred_element_type=jnp.float32)
                        + b1_r[...], 0.0)
        h = jnp.maximum(jnp.dot(h, w2t_r[...], preferred_element_type=jnp.float32)
                        + b2_r[...], 0.0)
        out_r[...] = jnp.dot(h, w3t_r[...],
                             preferred_element_type=jnp.float32) + b3_r[...]

    full = lambda shape: pl.BlockSpec(shape, lambda i: (0, 0))
    return pl.pallas_call(
        body,
        grid=(n // blk,),
        in_specs=[
            pl.BlockSpec((1, blk, DH), lambda i: (0, i, 0)),
            pl.BlockSpec((1, blk, DH), lambda i: (1, i, 0)),
            pl.BlockSpec((1, blk, L), lambda i: (0, i, 0)),
            pl.BlockSpec((1, blk, L), lambda i: (1, i, 0)),
            pl.BlockSpec((blk, d), lambda i: (i, 0)),
            full(wlt0.shape), full(wlt1.shape), full(bl.shape), full(wrt.shape),
            full(w1t.shape), full(b1.shape), full(w2t.shape), full(b2.shape),
            full(w3t.shape), full(b3.shape),
        ],
        out_specs=pl.BlockSpec((blk, 1), lambda i: (i, 0)),
        out_shape=jax.ShapeDtypeStruct((n, 1), jnp.float32),
    )(summed, summed, counts, counts, x, wlt0, wlt1, bl, wrt,
      w1t, b1, w2t, b2, w3t, b3)


def kernel(x, edge_index, Wl, bl, Wr, W1, b1, W2, b2, W3, b3):
    n, d = x.shape
    e = edge_index.shape[1]
    npad = _round_up(n + 1, NS * CHUNK)
    epad = _round_up(e, 2 * NS * GC)
    nchunk = epad // (NS * GC)
    epad2 = _round_up(e, NC * NS * CHUNK)
    nchunk2 = epad2 // (NC * NS * CHUNK)

    src = edge_index[0]
    dst = edge_index[1]

    xh0 = jnp.pad(x[:, :DH], ((0, npad - n), (0, 0)))
    xh1 = jnp.pad(x[:, DH:], ((0, npad - n), (0, 0)))
    xcomb = jnp.concatenate([xh0, xh1], axis=0)
    u = lax.bitcast_convert_type(xcomb, jnp.uint32)
    r16 = (u + 0x7FFF + ((u >> 16) & 1)) >> 16
    lo = r16[:, :PW]
    hi = r16[:, PW:]
    xpk = lax.bitcast_convert_type(lo | (hi << 16), jnp.int32)

    srcp = jnp.pad(src, (0, epad - e))
    dstp = jnp.pad(dst, (0, epad - e), constant_values=n)
    srcoff = jnp.stack([srcp, srcp + npad]).reshape(NC, NS, nchunk, GC)
    dstr = dstp.reshape(NS, nchunk, GC)

    dstp2 = jnp.pad(dst, (0, epad2 - e), constant_values=n)
    dstr2 = dstp2.reshape(NC, NS, nchunk2, CHUNK)

    summed = _sc_segment_sum(xpk, srcoff, dstr, npad, nchunk)
    counts = _sc_counts(dstr2, npad, nchunk2)

    wlt = Wl.T
    out = _tc_head(
        summed, counts, x,
        wlt[:DH], wlt[DH:], bl.reshape(1, -1), Wr.T,
        W1.T, b1.reshape(1, -1), W2.T, b2.reshape(1, -1),
        W3.T, b3.reshape(1, -1),
    )
    return out


# speedup vs baseline: 2.3042x; 1.0002x over previous
import functools

import jax
import jax.numpy as jnp
from jax import lax
from jax.experimental import pallas as pl
from jax.experimental.pallas import tpu as pltpu
from jax.experimental.pallas import tpu_sc as plsc

NC = 2
NS = 16
L = 16
CHUNK = 128
GC = 120
DH = 128
PW = DH // 2


def _round_up(a, b):
    return (a + b - 1) // b * b


def _sc_segment_sum(xpk, srcoff, dstr, npad, nchunk):
    mesh = plsc.VectorSubcoreMesh(core_axis_name="c", subcore_axis_name="s")
    rows_per_tile = npad // NS
    nzero = rows_per_tile // GC

    @functools.partial(
        pl.kernel,
        out_type=jax.ShapeDtypeStruct((NC, npad, DH), jnp.float32),
        mesh=mesh,
        scratch_types=[
            pltpu.VMEM((1, GC), jnp.int32),
            pltpu.VMEM((1, GC), jnp.int32),
            pltpu.VMEM((1, GC), jnp.int32),
            pltpu.VMEM((1, GC), jnp.int32),
            pltpu.VMEM((GC, PW), jnp.int32),
            pltpu.VMEM((GC, PW), jnp.int32),
            pltpu.VMEM((GC, DH), jnp.float32),
            pltpu.VMEM((GC, DH), jnp.float32),
            pltpu.VMEM_SHARED((npad, DH), jnp.float32),
            pltpu.SemaphoreType.DMA,
            pltpu.SemaphoreType.DMA,
            pltpu.SemaphoreType.DMA,
            pltpu.SemaphoreType.DMA,
        ],
        compiler_params=pltpu.CompilerParams(use_tc_tiling_on_sc=False),
    )
    def k(xpk_h, srcoff_h, dstr_h, out_h, srcs0, srcs1, dstb0, dstb1,
          raw0, raw1, fbuf0, fbuf1, acc, semg, semd, semsc, sems):
        c = lax.axis_index("c")
        s = lax.axis_index("s")

        def fill_g(i, _):
            for l in range(DH // L):
                fbuf0[i, pl.ds(l * L, L)] = jnp.zeros((L,), jnp.float32)
            return 0
        lax.fori_loop(0, GC, fill_g, 0)

        def zero_acc(j, _):
            base = s * rows_per_tile + j * GC
            pltpu.sync_copy(fbuf0, acc.at[pl.ds(base, GC)])
            return 0
        lax.fori_loop(0, nzero, zero_acc, 0)
        rem = rows_per_tile - nzero * GC
        if rem:
            pltpu.sync_copy(
                fbuf0.at[pl.ds(0, rem)],
                acc.at[pl.ds(s * rows_per_tile + nzero * GC, rem)])

        plsc.subcore_barrier()

        def wait_g(buf):
            pltpu.make_async_copy(xpk_h.at[pl.ds(0, GC)], buf, semg).wait()

        def wait_s(buf):
            pltpu.make_async_copy(
                srcoff_h.at[0, 0, pl.ds(0, 1)], buf, sems).wait()

        def wait_d(buf):
            pltpu.make_async_copy(
                dstr_h.at[0, pl.ds(0, 1)], buf, semd).wait()

        def wait_sc(buf):
            pltpu.make_async_copy(
                buf, acc.at[pl.ds(0, GC)], semsc).wait()

        def fire_src(j, sbuf):
            pltpu.async_copy(srcoff_h.at[c, s, pl.ds(j, 1)], sbuf, sems)

        def convert(raw, fbuf):
            def cv(i2, _):
                for r in range(4):
                    i = 4 * i2 + r
                    for g in range(PW // L):
                        w = raw[i, pl.ds(g * L, L)]
                        a0 = lax.bitcast_convert_type(
                            lax.shift_left(w, 16), jnp.float32)
                        a1 = lax.bitcast_convert_type(
                            w & jnp.int32(-65536), jnp.float32)
                        fbuf[i, pl.ds(L * g, L)] = a0
                        fbuf[i, pl.ds(PW + L * g, L)] = a1
                return 0
            lax.fori_loop(0, GC // 4, cv, 0)

        def stage(jq, srcs, raw, fbuf, dstb, first):
            wait_g(raw)
            if not first:
                wait_sc(fbuf)

            @pl.when(jq + 2 < nchunk)
            def _():
                fire_src(jq + 2, srcs)
            pltpu.async_copy(dstr_h.at[s, pl.ds(jq, 1)], dstb, semd)
            convert(raw, fbuf)

            @pl.when(jq + 2 < nchunk)
            def _():
                wait_s(srcs)
                pltpu.async_copy(xpk_h.at[srcs.at[0]], raw, semg)
            wait_d(dstb)
            pltpu.async_copy(fbuf, acc.at[dstb.at[0]], semsc, add=True)

        fire_src(0, srcs0)
        fire_src(1, srcs1)
        wait_s(srcs0)
        pltpu.async_copy(xpk_h.at[srcs0.at[0]], raw0, semg)
        wait_s(srcs1)
        pltpu.async_copy(xpk_h.at[srcs1.at[0]], raw1, semg)

        stage(0, srcs0, raw0, fbuf0, dstb0, True)
        stage(1, srcs1, raw1, fbuf1, dstb1, True)

        def body(jp, _):
            j0 = 2 * jp + 2
            stage(j0, srcs0, raw0, fbuf0, dstb0, False)
            stage(j0 + 1, srcs1, raw1, fbuf1, dstb1, False)
            return 0
        lax.fori_loop(0, nchunk // 2 - 1, body, 0)

        wait_sc(fbuf0)
        wait_sc(fbuf1)

        plsc.subcore_barrier()

        base = s * rows_per_tile
        pltpu.sync_copy(acc.at[pl.ds(base, rows_per_tile)],
                        out_h.at[c, pl.ds(base, rows_per_tile)])

    return k(xpk, srcoff, dstr)


def _sc_counts(dstr2, npad, nchunk2):
    mesh = plsc.VectorSubcoreMesh(core_axis_name="c", subcore_axis_name="s")
    rows_per_tile = npad // NS
    nzero = rows_per_tile // CHUNK

    @functools.partial(
        pl.kernel,
        out_type=jax.ShapeDtypeStruct((NC, npad, L), jnp.float32),
        mesh=mesh,
        scratch_types=[
            pltpu.VMEM((nchunk2, CHUNK), jnp.int32),
            pltpu.VMEM((CHUNK, L), jnp.float32),
            pltpu.VMEM_SHARED((npad, L), jnp.float32),
            pltpu.SemaphoreType.DMA,
        ],
        compiler_params=pltpu.CompilerParams(use_tc_tiling_on_sc=False),
    )
    def k(dstr_h, cnt_h, dstb, onesb, cacc, sem):
        c = lax.axis_index("c")
        s = lax.axis_index("s")

        def fill(i, _):
            onesb[i, :] = jnp.zeros((L,), jnp.float32)
            return 0
        lax.fori_loop(0, CHUNK, fill, 0)

        def zero_acc(j, _):
            base = s * rows_per_tile + j * CHUNK
            pltpu.sync_copy(onesb, cacc.at[pl.ds(base, CHUNK)])
            return 0
        lax.fori_loop(0, nzero, zero_acc, 0)

        def fill1(i, _):
            onesb[i, :] = jnp.ones((L,), jnp.float32)
            return 0
        lax.fori_loop(0, CHUNK, fill1, 0)

        plsc.subcore_barrier()

        pltpu.sync_copy(dstr_h.at[c, s], dstb)

        def body(j, _):
            pltpu.sync_copy(onesb, cacc.at[dstb.at[j]], add=True)
            return 0
        lax.fori_loop(0, nchunk2, body, 0)

        plsc.subcore_barrier()

        base = s * rows_per_tile
        pltpu.sync_copy(cacc.at[pl.ds(base, rows_per_tile)],
                        cnt_h.at[c, pl.ds(base, rows_per_tile)])

    return k(dstr2)


def _tc_head(summed, counts, x, wlt0, wlt1, bl, wrt, w1t, b1, w2t, b2,
             w3t, b3):
    n, d = x.shape
    blk = 2000

    def body(s0_r, s1_r, c0_r, c1_r, x_r, wlt0_r, wlt1_r, bl_r, wrt_r,
             w1t_r, b1_r, w2t_r, b2_r, w3t_r, b3_r, out_r):
        bf = jnp.bfloat16
        f32 = jnp.float32
        cnt = c0_r[0][:, 0:1] + c1_r[0][:, 0:1]
        rcp = 1.0 / jnp.maximum(cnt, 1.0)
        m0 = (s0_r[0] * rcp).astype(bf)
        m1 = (s1_r[0] * rcp).astype(bf)
        xv = x_r[...]
        conv = (jnp.dot(m0, wlt0_r[...].astype(bf), preferred_element_type=f32)
                + jnp.dot(m1, wlt1_r[...].astype(bf),
                          preferred_element_type=f32)
                + bl_r[...]
                + jnp.dot(xv.astype(bf), wrt_r[...].astype(bf),
                          preferred_element_type=f32))
        h = jnp.maximum(conv, 0.0) + xv
        h = jnp.maximum(jnp.dot(h.astype(bf), w1t_r[...].astype(bf),
                                preferred_element_type=f32)
                        + b1_r[...], 0.0)
        h = jnp.maximum(jnp.dot(h, w2t_r[...], preferred_element_type=f32)
                        + b2_r[...], 0.0)
        out_r[...] = jnp.dot(h, w3t_r[...],
                             preferred_element_type=f32) + b3_r[...]

    full = lambda shape: pl.BlockSpec(shape, lambda i: (0, 0))
    return pl.pallas_call(
        body,
        grid=(n // blk,),
        in_specs=[
            pl.BlockSpec((1, blk, DH), lambda i: (0, i, 0)),
            pl.BlockSpec((1, blk, DH), lambda i: (1, i, 0)),
            pl.BlockSpec((1, blk, L), lambda i: (0, i, 0)),
            pl.BlockSpec((1, blk, L), lambda i: (1, i, 0)),
            pl.BlockSpec((blk, d), lambda i: (i, 0)),
            full(wlt0.shape), full(wlt1.shape), full(bl.shape), full(wrt.shape),
            full(w1t.shape), full(b1.shape), full(w2t.shape), full(b2.shape),
            full(w3t.shape), full(b3.shape),
        ],
        out_specs=pl.BlockSpec((blk, 1), lambda i: (i, 0)),
        out_shape=jax.ShapeDtypeStruct((n, 1), jnp.float32),
    )(summed, summed, counts, counts, x, wlt0, wlt1, bl, wrt,
      w1t, b1, w2t, b2, w3t, b3)


def kernel(x, edge_index, Wl, bl, Wr, W1, b1, W2, b2, W3, b3):
    n, d = x.shape
    e = edge_index.shape[1]
    npad = _round_up(n + 1, NS * CHUNK)
    epad = _round_up(e, 2 * NS * GC)
    nchunk = epad // (NS * GC)
    epad2 = _round_up(e, NC * NS * CHUNK)
    nchunk2 = epad2 // (NC * NS * CHUNK)

    src = edge_index[0]
    dst = edge_index[1]

    xh0 = jnp.pad(x[:, :DH], ((0, npad - n), (0, 0)))
    xh1 = jnp.pad(x[:, DH:], ((0, npad - n), (0, 0)))
    xcomb = jnp.concatenate([xh0, xh1], axis=0)
    u = lax.bitcast_convert_type(xcomb, jnp.uint32)
    r16 = (u + 0x7FFF + ((u >> 16) & 1)) >> 16
    lo = r16[:, :PW]
    hi = r16[:, PW:]
    xpk = lax.bitcast_convert_type(lo | (hi << 16), jnp.int32)

    srcp = jnp.pad(src, (0, epad - e))
    dstp = jnp.pad(dst, (0, epad - e), constant_values=n)
    srcoff = jnp.stack([srcp, srcp + npad]).reshape(NC, NS, nchunk, GC)
    dstr = dstp.reshape(NS, nchunk, GC)

    dstp2 = jnp.pad(dst, (0, epad2 - e), constant_values=n)
    dstr2 = dstp2.reshape(NC, NS, nchunk2, CHUNK)

    summed = _sc_segment_sum(xpk, srcoff, dstr, npad, nchunk)
    counts = _sc_counts(dstr2, npad, nchunk2)

    wlt = Wl.T
    out = _tc_head(
        summed, counts, x,
        wlt[:DH], wlt[DH:], bl.reshape(1, -1), Wr.T,
        W1.T, b1.reshape(1, -1), W2.T, b2.reshape(1, -1),
        W3.T, b3.reshape(1, -1),
    )
    return out
